# Initial kernel scaffold; baseline (speedup 1.0000x reference)
import jax
import jax.numpy as jnp
from jax.experimental import pallas as pl


def kernel(x, edge_index, batch, W1, b1, W2, b2, W3, b3, Wl, bl):
    raise NotImplementedError("write your pallas kernel here")


# speedup vs baseline: 6.7696x; 6.7696x over previous
import functools

import jax
import jax.numpy as jnp
from jax import lax
from jax.experimental import pallas as pl
from jax.experimental.pallas import tpu as pltpu
from jax.experimental.pallas import tpu_sc as plsc

N = 10000
E = 320000
D = 128
H = 128
C = 16
G = 128

NB = 79
NP = NB * 128
NC, NS = 2, 16
NW = NC * NS
CHUNK = 128
CPW = 80
EPW = CPW * CHUNK
E2 = NW * EPW
DC = E2 // NS // CHUNK
STRIPE = NP // NS

_f32 = jnp.float32


def _deg_body(dst_hbm, z1_hbm, ones_hbm, deg_hbm, deg_sh, dstv, onesv, zv, sem):
    c = lax.axis_index("c")
    s = lax.axis_index("s")

    @pl.when(c == 0)
    def _():
        pltpu.sync_copy(z1_hbm, zv)
        pltpu.sync_copy(zv, deg_sh.at[pl.ds(s * STRIPE, STRIPE)])
        pltpu.sync_copy(dst_hbm.at[s], dstv)
        pltpu.sync_copy(ones_hbm, onesv)
        plsc.subcore_barrier()

        def step(i, carry):
            pltpu.sync_copy(onesv, deg_sh.at[dstv.at[i]], add=True)
            return carry

        lax.fori_loop(0, DC, step, 0)
        plsc.subcore_barrier()
        pltpu.sync_copy(deg_sh.at[pl.ds(s * STRIPE, STRIPE)], zv)
        pltpu.sync_copy(zv, deg_hbm.at[pl.ds(s * STRIPE, STRIPE)])


_deg_call = functools.partial(
    pl.kernel,
    out_type=jax.ShapeDtypeStruct((NP,), _f32),
    mesh=plsc.VectorSubcoreMesh(core_axis_name="c", subcore_axis_name="s"),
    scratch_types=[
        pltpu.VMEM_SHARED((NP,), _f32),
        pltpu.VMEM((DC, CHUNK), jnp.int32),
        pltpu.VMEM((CHUNK,), _f32),
        pltpu.VMEM((STRIPE,), _f32),
        pltpu.SemaphoreType.DMA,
    ],
)(_deg_body)


def _agg_body(hp_hbm, src_hbm, dst_hbm, z2_hbm, out_hbm,
              agg_sh, srcv, dstv, buf, sem):
    c = lax.axis_index("c")
    s = lax.axis_index("s")
    w = c * NS + s
    pltpu.sync_copy(z2_hbm, agg_sh.at[pl.ds(s * STRIPE, STRIPE)])
    pltpu.sync_copy(src_hbm.at[w], srcv)
    pltpu.sync_copy(dst_hbm.at[w], dstv)
    plsc.subcore_barrier()

    def step(i, carry):
        pltpu.async_copy(hp_hbm.at[srcv.at[i]], buf, sem).wait()
        pltpu.sync_copy(buf, agg_sh.at[dstv.at[i]], add=True)
        return carry

    lax.fori_loop(0, CPW, step, 0)
    plsc.subcore_barrier()
    pltpu.sync_copy(agg_sh.at[pl.ds(s * STRIPE, STRIPE)],
                    out_hbm.at[c, pl.ds(s * STRIPE, STRIPE)])


_agg_call = functools.partial(
    pl.kernel,
    out_type=jax.ShapeDtypeStruct((NC, NP, H), _f32),
    mesh=plsc.VectorSubcoreMesh(core_axis_name="c", subcore_axis_name="s"),
    scratch_types=[
        pltpu.VMEM_SHARED((NP, H), _f32),
        pltpu.VMEM((CPW, CHUNK), jnp.int32),
        pltpu.VMEM((CPW, CHUNK), jnp.int32),
        pltpu.VMEM((CHUNK, H), _f32),
        pltpu.SemaphoreType.DMA,
    ],
)(_agg_body)


def _mm1_body(x_ref, w_ref, deg_ref, hp_ref, dinv_ref):
    dinv = lax.rsqrt(deg_ref[...] + 1.0)
    dinv_ref[...] = dinv
    hp_ref[...] = jnp.dot(x_ref[...], w_ref[...],
                          preferred_element_type=_f32) * dinv


_mm1_call = pl.pallas_call(
    _mm1_body,
    grid=(NB,),
    in_specs=[
        pl.BlockSpec((128, D), lambda i: (i, 0)),
        pl.BlockSpec((D, H), lambda i: (0, 0)),
        pl.BlockSpec((128, 1), lambda i: (i, 0)),
    ],
    out_specs=[
        pl.BlockSpec((128, H), lambda i: (i, 0)),
        pl.BlockSpec((128, 1), lambda i: (i, 0)),
    ],
    out_shape=[
        jax.ShapeDtypeStruct((NP, H), _f32),
        jax.ShapeDtypeStruct((NP, 1), _f32),
    ],
)


def _mm2_body(a_ref, hp_ref, dinv_ref, b_ref, w_ref, hpn_ref):
    agg = a_ref[0] + a_ref[1] + hp_ref[...]
    h = jnp.maximum(agg * dinv_ref[...] + b_ref[...], 0.0)
    hpn_ref[...] = jnp.dot(h, w_ref[...],
                           preferred_element_type=_f32) * dinv_ref[...]


_mm2_call = pl.pallas_call(
    _mm2_body,
    grid=(NB,),
    in_specs=[
        pl.BlockSpec((2, 128, H), lambda i: (0, i, 0)),
        pl.BlockSpec((128, H), lambda i: (i, 0)),
        pl.BlockSpec((128, 1), lambda i: (i, 0)),
        pl.BlockSpec((1, H), lambda i: (0, 0)),
        pl.BlockSpec((H, H), lambda i: (0, 0)),
    ],
    out_specs=pl.BlockSpec((128, H), lambda i: (i, 0)),
    out_shape=jax.ShapeDtypeStruct((NP, H), _f32),
)


def _final_body(a_ref, hp_ref, dinv_ref, b_ref, batch_ref, wl_ref, bl_ref,
                out_ref, psum_ref, cnt_ref):
    i = pl.program_id(0)

    @pl.when(i == 0)
    def _():
        psum_ref[...] = jnp.zeros((G, H), _f32)
        cnt_ref[...] = jnp.zeros((G, 1), _f32)

    h3 = (a_ref[0] + a_ref[1] + hp_ref[...]) * dinv_ref[...] + b_ref[...]
    oh = (lax.broadcasted_iota(jnp.int32, (G, 128), 0)
          == batch_ref[0]).astype(_f32)
    psum_ref[...] += jnp.dot(oh, h3, preferred_element_type=_f32)
    cnt_ref[...] += jnp.sum(oh, axis=1, keepdims=True)

    @pl.when(i == NB - 1)
    def _():
        pooled = psum_ref[...] / jnp.maximum(cnt_ref[...], 1.0)
        out_ref[...] = jnp.dot(pooled, wl_ref[...],
                               preferred_element_type=_f32) + bl_ref[...]


_final_call = pl.pallas_call(
    _final_body,
    grid=(NB,),
    in_specs=[
        pl.BlockSpec((2, 128, H), lambda i: (0, i, 0)),
        pl.BlockSpec((128, H), lambda i: (i, 0)),
        pl.BlockSpec((128, 1), lambda i: (i, 0)),
        pl.BlockSpec((1, H), lambda i: (0, 0)),
        pl.BlockSpec((1, 1, 128), lambda i: (i, 0, 0)),
        pl.BlockSpec((H, C), lambda i: (0, 0)),
        pl.BlockSpec((1, C), lambda i: (0, 0)),
    ],
    out_specs=pl.BlockSpec((G, C), lambda i: (0, 0)),
    out_shape=jax.ShapeDtypeStruct((G, C), _f32),
    scratch_shapes=[
        pltpu.VMEM((G, H), _f32),
        pltpu.VMEM((G, 1), _f32),
    ],
)


def kernel(x, edge_index, batch, W1, b1, W2, b2, W3, b3, Wl, bl):
    src = edge_index[0]
    dst = edge_index[1]
    pad = E2 - E
    srcp = jnp.concatenate([src, jnp.zeros((pad,), jnp.int32)])
    dstp = jnp.concatenate([dst, jnp.full((pad,), N, jnp.int32)])
    src3 = srcp.reshape(NW, CPW, CHUNK)
    dst3 = dstp.reshape(NW, CPW, CHUNK)
    dstd = dstp.reshape(NS, DC, CHUNK)
    xp = jnp.pad(x, ((0, NP - N), (0, 0)))
    batchp = jnp.concatenate(
        [batch, jnp.full((NP - N,), G, jnp.int32)]).reshape(NB, 1, 128)
    z1 = jnp.zeros((STRIPE,), _f32)
    z2 = jnp.zeros((STRIPE, 128), _f32)
    ones1 = jnp.ones((CHUNK,), _f32)

    deg = _deg_call(dstd, z1, ones1)
    hp1, dinv = _mm1_call(xp, W1, deg.reshape(NP, 1))
    a1 = _agg_call(hp1, src3, dst3, z2)
    hp2 = _mm2_call(a1, hp1, dinv, b1.reshape(1, H), W2)
    a2 = _agg_call(hp2, src3, dst3, z2)
    hp3 = _mm2_call(a2, hp2, dinv, b2.reshape(1, H), W3)
    a3 = _agg_call(hp3, src3, dst3, z2)
    return _final_call(a3, hp3, dinv, b3.reshape(1, H), batchp,
                       Wl, bl.reshape(1, C))


# speedup vs baseline: 7.5446x; 1.1145x over previous
import functools

import jax
import jax.numpy as jnp
from jax import lax
from jax.experimental import pallas as pl
from jax.experimental.pallas import tpu as pltpu
from jax.experimental.pallas import tpu_sc as plsc

N = 10000
E = 320000
D = 128
H = 128
C = 16
G = 128

NB = 79
NP = NB * 128
NC, NS = 2, 16
NW = NC * NS
ACH = 128
NCH = 80
EPW = NCH * ACH
E2 = NW * EPW
DCH = 128
DC = E2 // NS // DCH
STRIPE = NP // NS

_f32 = jnp.float32


def _deg_body(dst_hbm, z1_hbm, ones_hbm, deg_hbm, deg_sh, dstv, onesv, zv, sem):
    c = lax.axis_index("c")
    s = lax.axis_index("s")

    @pl.when(c == 0)
    def _():
        pltpu.sync_copy(z1_hbm, zv)
        pltpu.sync_copy(zv, deg_sh.at[pl.ds(s * STRIPE, STRIPE)])
        pltpu.sync_copy(dst_hbm.at[s], dstv)
        pltpu.sync_copy(ones_hbm, onesv)
        plsc.subcore_barrier()

        def step(i, carry):
            pltpu.sync_copy(onesv, deg_sh.at[dstv.at[i]], add=True)
            return carry

        lax.fori_loop(0, DC, step, 0)
        plsc.subcore_barrier()
        pltpu.sync_copy(deg_sh.at[pl.ds(s * STRIPE, STRIPE)], zv)
        pltpu.sync_copy(zv, deg_hbm.at[pl.ds(s * STRIPE, STRIPE)])


_deg_call = functools.partial(
    pl.kernel,
    out_type=jax.ShapeDtypeStruct((NP,), _f32),
    mesh=plsc.VectorSubcoreMesh(core_axis_name="c", subcore_axis_name="s"),
    scratch_types=[
        pltpu.VMEM_SHARED((NP,), _f32),
        pltpu.VMEM((DC, DCH), jnp.int32),
        pltpu.VMEM((DCH,), _f32),
        pltpu.VMEM((STRIPE,), _f32),
        pltpu.SemaphoreType.DMA,
    ],
)(_deg_body)


def _agg_body(hp_hbm, src_hbm, dst_hbm, z2_hbm, out_hbm,
              agg_sh, srcv, didx0, didx1, buf0, buf1,
              sem0, sem1, dsem0, dsem1):
    c = lax.axis_index("c")
    s = lax.axis_index("s")
    w = c * NS + s
    pltpu.sync_copy(z2_hbm, agg_sh.at[pl.ds(s * STRIPE, STRIPE)])
    pltpu.sync_copy(src_hbm.at[w], srcv)
    plsc.subcore_barrier()

    pltpu.async_copy(hp_hbm.at[srcv.at[0]], buf0, sem0)
    pltpu.async_copy(hp_hbm.at[srcv.at[1]], buf1, sem1)
    pltpu.async_copy(dst_hbm.at[w, 0], didx0, dsem0)
    pltpu.async_copy(dst_hbm.at[w, 1], didx1, dsem1)

    def step(g, carry):
        bufs = ((buf0, sem0, didx0, dsem0), (buf1, sem1, didx1, dsem1))
        for b, (buf, sem, didx, dsem) in enumerate(bufs):
            ci = 2 * g + b
            pltpu.make_async_copy(hp_hbm.at[srcv.at[ci]], buf, sem).wait()
            pltpu.make_async_copy(dst_hbm.at[w, ci], didx, dsem).wait()
            pltpu.sync_copy(buf, agg_sh.at[didx], add=True)

            @pl.when(ci + 2 < NCH)
            def _():
                pltpu.async_copy(hp_hbm.at[srcv.at[ci + 2]], buf, sem)
                pltpu.async_copy(dst_hbm.at[w, ci + 2], didx, dsem)

        return carry

    lax.fori_loop(0, NCH // 2, step, 0)
    plsc.subcore_barrier()
    pltpu.sync_copy(agg_sh.at[pl.ds(s * STRIPE, STRIPE)],
                    out_hbm.at[c, pl.ds(s * STRIPE, STRIPE)])


_agg_call = functools.partial(
    pl.kernel,
    out_type=jax.ShapeDtypeStruct((NC, NP, H), _f32),
    mesh=plsc.VectorSubcoreMesh(core_axis_name="c", subcore_axis_name="s"),
    scratch_types=[
        pltpu.VMEM_SHARED((NP, H), _f32),
        pltpu.VMEM((NCH, ACH), jnp.int32),
        pltpu.VMEM((ACH,), jnp.int32),
        pltpu.VMEM((ACH,), jnp.int32),
        pltpu.VMEM((ACH, H), _f32),
        pltpu.VMEM((ACH, H), _f32),
        pltpu.SemaphoreType.DMA,
        pltpu.SemaphoreType.DMA,
        pltpu.SemaphoreType.DMA,
        pltpu.SemaphoreType.DMA,
    ],
)(_agg_body)


def _mm1_body(x_ref, w_ref, deg_ref, hp_ref, dinv_ref):
    dinv = lax.rsqrt(deg_ref[...] + 1.0)
    dinv_ref[...] = dinv
    hp_ref[...] = jnp.dot(x_ref[...], w_ref[...],
                          preferred_element_type=_f32) * dinv


_mm1_call = pl.pallas_call(
    _mm1_body,
    grid=(NB,),
    in_specs=[
        pl.BlockSpec((128, D), lambda i: (i, 0)),
        pl.BlockSpec((D, H), lambda i: (0, 0)),
        pl.BlockSpec((128, 1), lambda i: (i, 0)),
    ],
    out_specs=[
        pl.BlockSpec((128, H), lambda i: (i, 0)),
        pl.BlockSpec((128, 1), lambda i: (i, 0)),
    ],
    out_shape=[
        jax.ShapeDtypeStruct((NP, H), _f32),
        jax.ShapeDtypeStruct((NP, 1), _f32),
    ],
)


def _mm2_body(a_ref, hp_ref, dinv_ref, b_ref, w_ref, hpn_ref):
    agg = a_ref[0] + a_ref[1] + hp_ref[...]
    h = jnp.maximum(agg * dinv_ref[...] + b_ref[...], 0.0)
    hpn_ref[...] = jnp.dot(h, w_ref[...],
                           preferred_element_type=_f32) * dinv_ref[...]


_mm2_call = pl.pallas_call(
    _mm2_body,
    grid=(NB,),
    in_specs=[
        pl.BlockSpec((2, 128, H), lambda i: (0, i, 0)),
        pl.BlockSpec((128, H), lambda i: (i, 0)),
        pl.BlockSpec((128, 1), lambda i: (i, 0)),
        pl.BlockSpec((1, H), lambda i: (0, 0)),
        pl.BlockSpec((H, H), lambda i: (0, 0)),
    ],
    out_specs=pl.BlockSpec((128, H), lambda i: (i, 0)),
    out_shape=jax.ShapeDtypeStruct((NP, H), _f32),
)


def _final_body(a_ref, hp_ref, dinv_ref, b_ref, batch_ref, wl_ref, bl_ref,
                out_ref, psum_ref, cnt_ref):
    i = pl.program_id(0)

    @pl.when(i == 0)
    def _():
        psum_ref[...] = jnp.zeros((G, H), _f32)
        cnt_ref[...] = jnp.zeros((G, 1), _f32)

    h3 = (a_ref[0] + a_ref[1] + hp_ref[...]) * dinv_ref[...] + b_ref[...]
    oh = (lax.broadcasted_iota(jnp.int32, (G, 128), 0)
          == batch_ref[0]).astype(_f32)
    psum_ref[...] += jnp.dot(oh, h3, preferred_element_type=_f32)
    cnt_ref[...] += jnp.sum(oh, axis=1, keepdims=True)

    @pl.when(i == NB - 1)
    def _():
        pooled = psum_ref[...] / jnp.maximum(cnt_ref[...], 1.0)
        out_ref[...] = jnp.dot(pooled, wl_ref[...],
                               preferred_element_type=_f32) + bl_ref[...]


_final_call = pl.pallas_call(
    _final_body,
    grid=(NB,),
    in_specs=[
        pl.BlockSpec((2, 128, H), lambda i: (0, i, 0)),
        pl.BlockSpec((128, H), lambda i: (i, 0)),
        pl.BlockSpec((128, 1), lambda i: (i, 0)),
        pl.BlockSpec((1, H), lambda i: (0, 0)),
        pl.BlockSpec((1, 1, 128), lambda i: (i, 0, 0)),
        pl.BlockSpec((H, C), lambda i: (0, 0)),
        pl.BlockSpec((1, C), lambda i: (0, 0)),
    ],
    out_specs=pl.BlockSpec((G, C), lambda i: (0, 0)),
    out_shape=jax.ShapeDtypeStruct((G, C), _f32),
    scratch_shapes=[
        pltpu.VMEM((G, H), _f32),
        pltpu.VMEM((G, 1), _f32),
    ],
)


def kernel(x, edge_index, batch, W1, b1, W2, b2, W3, b3, Wl, bl):
    src = edge_index[0]
    dst = edge_index[1]
    pad = E2 - E
    srcp = jnp.concatenate([src, jnp.zeros((pad,), jnp.int32)])
    dstp = jnp.concatenate([dst, jnp.full((pad,), N, jnp.int32)])
    src3 = srcp.reshape(NW, NCH, ACH)
    dst3 = dstp.reshape(NW, NCH, ACH)
    dstd = dstp.reshape(NS, DC, DCH)
    xp = jnp.pad(x, ((0, NP - N), (0, 0)))
    batchp = jnp.concatenate(
        [batch, jnp.full((NP - N,), G, jnp.int32)]).reshape(NB, 1, 128)
    z1 = jnp.zeros((STRIPE,), _f32)
    z2 = jnp.zeros((STRIPE, 128), _f32)
    ones1 = jnp.ones((DCH,), _f32)

    deg = _deg_call(dstd, z1, ones1)
    hp1, dinv = _mm1_call(xp, W1, deg.reshape(NP, 1))
    a1 = _agg_call(hp1, src3, dst3, z2)
    hp2 = _mm2_call(a1, hp1, dinv, b1.reshape(1, H), W2)
    a2 = _agg_call(hp2, src3, dst3, z2)
    hp3 = _mm2_call(a2, hp2, dinv, b2.reshape(1, H), W3)
    a3 = _agg_call(hp3, src3, dst3, z2)
    return _final_call(a3, hp3, dinv, b3.reshape(1, H), batchp,
                       Wl, bl.reshape(1, C))


# speedup vs baseline: 21.0104x; 2.7848x over previous
import functools

import jax
import jax.numpy as jnp
from jax import lax
from jax.experimental import pallas as pl
from jax.experimental.pallas import tpu as pltpu
from jax.experimental.pallas import tpu_sc as plsc

N = 10000
E = 320000
D = 128
H = 128
C = 16
G = 128

NB = 79
NP = NB * 128
NC, NS = 2, 16
NW = NC * NS
ACH = 128
NCH = 80
EPW = NCH * ACH
E2 = NW * EPW
DCH = 128
DC = E2 // NS // DCH
STRIPE = NP // NS

_f32 = jnp.float32


def _deg_body(dst_hbm, z1_hbm, ones_hbm, deg_hbm, deg_sh, dstv, onesv, zv, sem):
    c = lax.axis_index("c")
    s = lax.axis_index("s")

    @pl.when(c == 0)
    def _():
        pltpu.sync_copy(z1_hbm, zv)
        pltpu.sync_copy(zv, deg_sh.at[pl.ds(s * STRIPE, STRIPE)])
        pltpu.sync_copy(dst_hbm.at[s], dstv)
        pltpu.sync_copy(ones_hbm, onesv)
        plsc.subcore_barrier()

        def step(i, carry):
            pltpu.sync_copy(onesv, deg_sh.at[dstv.at[i]], add=True)
            return carry

        nch = jnp.where(s == NS - 1, (E - (NS - 1) * (E2 // NS)) // DCH, DC)
        lax.fori_loop(0, nch, step, 0)
        plsc.subcore_barrier()
        pltpu.sync_copy(deg_sh.at[pl.ds(s * STRIPE, STRIPE)], zv)
        pltpu.sync_copy(zv, deg_hbm.at[pl.ds(s * STRIPE, STRIPE)])


_deg_call = functools.partial(
    pl.kernel,
    out_type=jax.ShapeDtypeStruct((NP,), _f32),
    mesh=plsc.VectorSubcoreMesh(core_axis_name="c", subcore_axis_name="s"),
    scratch_types=[
        pltpu.VMEM_SHARED((NP,), _f32),
        pltpu.VMEM((DC, DCH), jnp.int32),
        pltpu.VMEM((DCH,), _f32),
        pltpu.VMEM((STRIPE,), _f32),
        pltpu.SemaphoreType.DMA,
    ],
)(_deg_body)


def _agg_body(hp_hbm, src_hbm, dst_hbm, z2_hbm, out_hbm,
              agg_sh, srcv, didx0, didx1, buf0, buf1,
              sem0, sem1, dsem0, dsem1):
    c = lax.axis_index("c")
    s = lax.axis_index("s")
    w = c * NS + s
    pltpu.sync_copy(z2_hbm, agg_sh.at[pl.ds(s * STRIPE, STRIPE)])
    pltpu.sync_copy(src_hbm.at[w], srcv)
    plsc.subcore_barrier()

    pltpu.async_copy(hp_hbm.at[srcv.at[0]], buf0, sem0)
    pltpu.async_copy(hp_hbm.at[srcv.at[1]], buf1, sem1)
    pltpu.async_copy(dst_hbm.at[w, 0], didx0, dsem0)
    pltpu.async_copy(dst_hbm.at[w, 1], didx1, dsem1)

    nch = jnp.where(w == NW - 1, (E - (NW - 1) * EPW) // ACH, NCH)

    def step(g, carry):
        bufs = ((buf0, sem0, didx0, dsem0), (buf1, sem1, didx1, dsem1))
        for b, (buf, sem, didx, dsem) in enumerate(bufs):
            ci = 2 * g + b
            pltpu.make_async_copy(hp_hbm.at[srcv.at[ci]], buf, sem).wait()
            pltpu.make_async_copy(dst_hbm.at[w, ci], didx, dsem).wait()
            pltpu.sync_copy(buf, agg_sh.at[didx], add=True)

            @pl.when(ci + 2 < nch)
            def _():
                pltpu.async_copy(hp_hbm.at[srcv.at[ci + 2]], buf, sem)
                pltpu.async_copy(dst_hbm.at[w, ci + 2], didx, dsem)

        return carry

    lax.fori_loop(0, nch // 2, step, 0)
    plsc.subcore_barrier()
    pltpu.sync_copy(agg_sh.at[pl.ds(s * STRIPE, STRIPE)],
                    out_hbm.at[c, pl.ds(s * STRIPE, STRIPE)])


_agg_call = functools.partial(
    pl.kernel,
    out_type=jax.ShapeDtypeStruct((NC, NP, H), _f32),
    mesh=plsc.VectorSubcoreMesh(core_axis_name="c", subcore_axis_name="s"),
    scratch_types=[
        pltpu.VMEM_SHARED((NP, H), _f32),
        pltpu.VMEM((NCH, ACH), jnp.int32),
        pltpu.VMEM((ACH,), jnp.int32),
        pltpu.VMEM((ACH,), jnp.int32),
        pltpu.VMEM((ACH, H), _f32),
        pltpu.VMEM((ACH, H), _f32),
        pltpu.SemaphoreType.DMA,
        pltpu.SemaphoreType.DMA,
        pltpu.SemaphoreType.DMA,
        pltpu.SemaphoreType.DMA,
    ],
)(_agg_body)


def _mm1_body(x_ref, w_ref, deg_ref, hp_ref, dinv_ref):
    dinv = lax.rsqrt(deg_ref[...] + 1.0)
    dinv_ref[...] = dinv
    hp_ref[...] = jnp.dot(x_ref[...], w_ref[...],
                          preferred_element_type=_f32) * dinv


_mm1_call = pl.pallas_call(
    _mm1_body,
    grid=(NB,),
    in_specs=[
        pl.BlockSpec((128, D), lambda i: (i, 0)),
        pl.BlockSpec((D, H), lambda i: (0, 0)),
        pl.BlockSpec((128, 1), lambda i: (i, 0)),
    ],
    out_specs=[
        pl.BlockSpec((128, H), lambda i: (i, 0)),
        pl.BlockSpec((128, 1), lambda i: (i, 0)),
    ],
    out_shape=[
        jax.ShapeDtypeStruct((NP, H), _f32),
        jax.ShapeDtypeStruct((NP, 1), _f32),
    ],
)


def _mm2_body(a_ref, hp_ref, dinv_ref, b_ref, w_ref, hpn_ref):
    agg = a_ref[0] + a_ref[1] + hp_ref[...]
    h = jnp.maximum(agg * dinv_ref[...] + b_ref[...], 0.0)
    hpn_ref[...] = jnp.dot(h, w_ref[...],
                           preferred_element_type=_f32) * dinv_ref[...]


_mm2_call = pl.pallas_call(
    _mm2_body,
    grid=(NB,),
    in_specs=[
        pl.BlockSpec((2, 128, H), lambda i: (0, i, 0)),
        pl.BlockSpec((128, H), lambda i: (i, 0)),
        pl.BlockSpec((128, 1), lambda i: (i, 0)),
        pl.BlockSpec((1, H), lambda i: (0, 0)),
        pl.BlockSpec((H, H), lambda i: (0, 0)),
    ],
    out_specs=pl.BlockSpec((128, H), lambda i: (i, 0)),
    out_shape=jax.ShapeDtypeStruct((NP, H), _f32),
)


def _final_body(a_ref, hp_ref, dinv_ref, b_ref, batch_ref, wl_ref, bl_ref,
                out_ref, psum_ref, cnt_ref):
    i = pl.program_id(0)

    @pl.when(i == 0)
    def _():
        psum_ref[...] = jnp.zeros((G, H), _f32)
        cnt_ref[...] = jnp.zeros((G, 1), _f32)

    h3 = (a_ref[0] + a_ref[1] + hp_ref[...]) * dinv_ref[...] + b_ref[...]
    oh = (lax.broadcasted_iota(jnp.int32, (G, 128), 0)
          == batch_ref[0]).astype(_f32)
    psum_ref[...] += jnp.dot(oh, h3, preferred_element_type=_f32)
    cnt_ref[...] += jnp.sum(oh, axis=1, keepdims=True)

    @pl.when(i == NB - 1)
    def _():
        pooled = psum_ref[...] / jnp.maximum(cnt_ref[...], 1.0)
        out_ref[...] = jnp.dot(pooled, wl_ref[...],
                               preferred_element_type=_f32) + bl_ref[...]


_final_call = pl.pallas_call(
    _final_body,
    grid=(NB,),
    in_specs=[
        pl.BlockSpec((2, 128, H), lambda i: (0, i, 0)),
        pl.BlockSpec((128, H), lambda i: (i, 0)),
        pl.BlockSpec((128, 1), lambda i: (i, 0)),
        pl.BlockSpec((1, H), lambda i: (0, 0)),
        pl.BlockSpec((1, 1, 128), lambda i: (i, 0, 0)),
        pl.BlockSpec((H, C), lambda i: (0, 0)),
        pl.BlockSpec((1, C), lambda i: (0, 0)),
    ],
    out_specs=pl.BlockSpec((G, C), lambda i: (0, 0)),
    out_shape=jax.ShapeDtypeStruct((G, C), _f32),
    scratch_shapes=[
        pltpu.VMEM((G, H), _f32),
        pltpu.VMEM((G, 1), _f32),
    ],
)


def kernel(x, edge_index, batch, W1, b1, W2, b2, W3, b3, Wl, bl):
    src = edge_index[0]
    dst = edge_index[1]
    pad = E2 - E
    srcp = jnp.concatenate([src, jnp.zeros((pad,), jnp.int32)])
    dstp = jnp.concatenate([dst, jnp.full((pad,), N, jnp.int32)])
    src3 = srcp.reshape(NW, NCH, ACH)
    dst3 = dstp.reshape(NW, NCH, ACH)
    dstd = dstp.reshape(NS, DC, DCH)
    xp = jnp.pad(x, ((0, NP - N), (0, 0)))
    batchp = jnp.concatenate(
        [batch, jnp.full((NP - N,), G, jnp.int32)]).reshape(NB, 1, 128)
    z1 = jnp.zeros((STRIPE,), _f32)
    z2 = jnp.zeros((STRIPE, 128), _f32)
    ones1 = jnp.ones((DCH,), _f32)

    deg = _deg_call(dstd, z1, ones1)
    hp1, dinv = _mm1_call(xp, W1, deg.reshape(NP, 1))
    a1 = _agg_call(hp1, src3, dst3, z2)
    hp2 = _mm2_call(a1, hp1, dinv, b1.reshape(1, H), W2)
    a2 = _agg_call(hp2, src3, dst3, z2)
    hp3 = _mm2_call(a2, hp2, dinv, b2.reshape(1, H), W3)
    a3 = _agg_call(hp3, src3, dst3, z2)
    return _final_call(a3, hp3, dinv, b3.reshape(1, H), batchp,
                       Wl, bl.reshape(1, C))
